# baseline (speedup 1.0000x reference)
import functools

import jax
import jax.numpy as jnp
from jax import lax
from jax.experimental import pallas as pl
from jax.experimental.pallas import tpu as pltpu
from jax.experimental.pallas import tpu_sc as plsc

VOCAB = 1000000
K = 64
N_CLASSES = 2
B = 4096
L = 200

NC = 2
NS = 16
NW = NC * NS
ROWS = B // NW
VBLK = 40960
DEPTH = 10


def _proj_body(w_ref, embt_ref, pp_ref):
  pt = jnp.dot(w_ref[...], embt_ref[...], preferred_element_type=jnp.float32)
  lo = lax.bitcast_convert_type(
      pt[0].astype(jnp.bfloat16), jnp.uint16).astype(jnp.uint32)
  hi = lax.bitcast_convert_type(
      pt[1].astype(jnp.bfloat16), jnp.uint16).astype(jnp.uint32)
  pp_ref[...] = lo | (hi << 16)


def _project(w, embt):
  grid = (VOCAB + VBLK - 1) // VBLK
  return pl.pallas_call(
      _proj_body,
      grid=(grid,),
      in_specs=[
          pl.BlockSpec((N_CLASSES, K), lambda i: (0, 0)),
          pl.BlockSpec((K, VBLK), lambda i: (0, i)),
      ],
      out_specs=pl.BlockSpec((VBLK,), lambda i: (i,)),
      out_shape=jax.ShapeDtypeStruct((VOCAB,), jnp.uint32),
  )(w, embt)


def _pool_body(xt_hbm, pp_hbm, bt_hbm, out_hbm, xv, v, b_v, ob, shared,
               *sems):
  sid = lax.axis_index("s")
  wid = sid * NC + lax.axis_index("c")
  base = wid * ROWS

  for c in range(NS):
    off = c * 64000
    n = 64000 if c < NS - 1 else VOCAB - 15 * 64000

    @pl.when(sid == c)
    def _():
      pltpu.sync_copy(pp_hbm.at[pl.ds(off, n)], shared.at[pl.ds(off, n)])

  pltpu.sync_copy(xt_hbm.at[:, pl.ds(base, ROWS)], xv)
  pltpu.sync_copy(bt_hbm, b_v)
  plsc.subcore_barrier()

  def fire(l, s):
    pltpu.async_copy(shared.at[xv.at[l]], v.at[s], sems[s])

  def wait(l, s):
    pltpu.make_async_copy(shared.at[xv.at[l]], v.at[s], sems[s]).wait()

  for s in range(DEPTH):
    fire(s, s)

  zero = jnp.zeros((16,), jnp.float32)
  himask = jnp.full((16,), 0xFFFF0000, jnp.uint32)

  def grp_step(g, accs):
    accs = list(accs)
    for s in range(DEPTH):
      l = g * DEPTH + s
      wait(l, s)
      vals = []
      for k in range(8):
        packed = v[s, pl.ds(16 * k, 16)]
        vals.append(lax.bitcast_convert_type(packed << 16, jnp.float32))
        vals.append(lax.bitcast_convert_type(packed & himask, jnp.float32))

      @pl.when(l + DEPTH < L)
      def _():
        fire(l + DEPTH, s)

      for k in range(16):
        accs[k] = accs[k] + vals[k]
    return tuple(accs)

  accs = lax.fori_loop(0, L // DEPTH, grp_step, (zero,) * 16)

  b0 = b_v[pl.ds(0, 16)]
  b1 = b_v[pl.ds(16, 16)]
  lanes = jnp.arange(16, dtype=jnp.int32)
  inv_l = jnp.float32(1.0 / L)
  ones = jnp.ones((16,), jnp.bool_)
  for k in range(8):
    pos = 32 * k + 2 * lanes
    plsc.store_scatter(ob, [pos], accs[2 * k] * inv_l + b0, mask=ones)
    plsc.store_scatter(ob, [pos + 1], accs[2 * k + 1] * inv_l + b1, mask=ones)

  pltpu.sync_copy(ob, out_hbm.at[pl.ds(base * N_CLASSES, ROWS * N_CLASSES)])


def _pool(xt, pp, bt):
  mesh = plsc.VectorSubcoreMesh(core_axis_name="c", subcore_axis_name="s")
  return pl.kernel(
      _pool_body,
      out_type=jax.ShapeDtypeStruct((B * N_CLASSES,), jnp.float32),
      mesh=mesh,
      scratch_types=[
          pltpu.VMEM((L, ROWS), jnp.int32),
          pltpu.VMEM((DEPTH, ROWS), jnp.uint32),
          pltpu.VMEM((2 * 16,), jnp.float32),
          pltpu.VMEM((ROWS * N_CLASSES,), jnp.float32),
          pltpu.VMEM_SHARED((VOCAB,), jnp.uint32),
      ] + [pltpu.SemaphoreType.DMA] * DEPTH,
      compiler_params=pltpu.CompilerParams(
          needs_layout_passes=False, use_tc_tiling_on_sc=False),
  )(xt, pp, bt)


@jax.jit
def _run(xt, embt, w, bt):
  pp = _project(w, embt)
  return _pool(xt, pp, bt).reshape(B, N_CLASSES)


def kernel(x, emb, W, b):
  bt = jnp.repeat(b, 16)
  return _run(x.astype(jnp.int32).T, emb.T, W, bt)


# speedup vs baseline: 1.0048x; 1.0048x over previous
import jax
import jax.numpy as jnp
from jax import lax
from jax.experimental import pallas as pl
from jax.experimental.pallas import tpu as pltpu
from jax.experimental.pallas import tpu_sc as plsc

VOCAB = 1000000
K = 64
N_CLASSES = 2
B = 4096
L = 200

NC = 2
NS = 16
NW = NC * NS
ROWS = B // NW
VBLK = 32768
DEPTH = 10


def _proj_body(w_ref, embt_ref, pp_ref):
  pt = jnp.dot(w_ref[...], embt_ref[...], preferred_element_type=jnp.float32)
  lo = lax.bitcast_convert_type(
      pt[0].astype(jnp.bfloat16), jnp.uint16).astype(jnp.uint32)
  hi = lax.bitcast_convert_type(
      pt[1].astype(jnp.bfloat16), jnp.uint16).astype(jnp.uint32)
  pp_ref[...] = lo | (hi << 16)


def _project(w, embt):
  grid = (VOCAB + VBLK - 1) // VBLK
  return pl.pallas_call(
      _proj_body,
      grid=(grid,),
      in_specs=[
          pl.BlockSpec((N_CLASSES, K), lambda i: (0, 0)),
          pl.BlockSpec((K, VBLK), lambda i: (0, i)),
      ],
      out_specs=pl.BlockSpec((VBLK,), lambda i: (i,)),
      out_shape=jax.ShapeDtypeStruct((VOCAB,), jnp.uint32),
  )(w, embt)


def _pool_body(xt_hbm, pp_hbm, bt_hbm, out_hbm, xv, v, b_v, ob, shared,
               *sems):
  sid = lax.axis_index("s")
  wid = sid * NC + lax.axis_index("c")
  base = wid * ROWS

  for c in range(NS):
    off = c * 64000
    n = 64000 if c < NS - 1 else VOCAB - 15 * 64000

    @pl.when(sid == c)
    def _():
      pltpu.sync_copy(pp_hbm.at[pl.ds(off, n)], shared.at[pl.ds(off, n)])

  pltpu.sync_copy(xt_hbm.at[:, pl.ds(base, ROWS)], xv)
  pltpu.sync_copy(bt_hbm, b_v)
  plsc.subcore_barrier()

  def fire(l, s):
    pltpu.async_copy(shared.at[xv.at[l]], v.at[s], sems[s])

  def wait(l, s):
    pltpu.make_async_copy(shared.at[xv.at[l]], v.at[s], sems[s]).wait()

  for s in range(DEPTH):
    fire(s, s)

  zero = jnp.zeros((16,), jnp.float32)
  himask = jnp.full((16,), 0xFFFF0000, jnp.uint32)

  def grp_step(g, accs):
    accs = list(accs)
    for s in range(DEPTH):
      l = g * DEPTH + s
      wait(l, s)
      vals = []
      for k in range(8):
        packed = v[s, pl.ds(16 * k, 16)]
        vals.append(lax.bitcast_convert_type(packed << 16, jnp.float32))
        vals.append(lax.bitcast_convert_type(packed & himask, jnp.float32))

      @pl.when(l + DEPTH < L)
      def _():
        fire(l + DEPTH, s)

      for k in range(16):
        accs[k] = accs[k] + vals[k]
    return tuple(accs)

  accs = lax.fori_loop(0, L // DEPTH, grp_step, (zero,) * 16)

  b0 = b_v[pl.ds(0, 16)]
  b1 = b_v[pl.ds(16, 16)]
  lanes = jnp.arange(16, dtype=jnp.int32)
  inv_l = jnp.float32(1.0 / L)
  ones = jnp.ones((16,), jnp.bool_)
  for k in range(8):
    pos = 32 * k + 2 * lanes
    plsc.store_scatter(ob, [pos], accs[2 * k] * inv_l + b0, mask=ones)
    plsc.store_scatter(ob, [pos + 1], accs[2 * k + 1] * inv_l + b1, mask=ones)

  pltpu.sync_copy(ob, out_hbm.at[pl.ds(base * N_CLASSES, ROWS * N_CLASSES)])


def _pool(xt, pp, bt):
  mesh = plsc.VectorSubcoreMesh(core_axis_name="c", subcore_axis_name="s")
  return pl.kernel(
      _pool_body,
      out_type=jax.ShapeDtypeStruct((B * N_CLASSES,), jnp.float32),
      mesh=mesh,
      scratch_types=[
          pltpu.VMEM((L, ROWS), jnp.int32),
          pltpu.VMEM((DEPTH, ROWS), jnp.uint32),
          pltpu.VMEM((2 * 16,), jnp.float32),
          pltpu.VMEM((ROWS * N_CLASSES,), jnp.float32),
          pltpu.VMEM_SHARED((VOCAB,), jnp.uint32),
      ] + [pltpu.SemaphoreType.DMA] * DEPTH,
      compiler_params=pltpu.CompilerParams(
          needs_layout_passes=False, use_tc_tiling_on_sc=False),
  )(xt, pp, bt)


@jax.jit
def _run(xt, embt, w, bt):
  pp = _project(w, embt)
  return _pool(xt, pp, bt).reshape(B, N_CLASSES)


def kernel(x, emb, W, b):
  bt = jnp.repeat(b, 16)
  return _run(x.astype(jnp.int32).T, emb.T, W, bt)
